# baseline (speedup 1.0000x reference)
import functools

import jax
import jax.numpy as jnp
from jax import lax
from jax.experimental import pallas as pl
from jax.experimental.pallas import tpu as pltpu
from jax.experimental.pallas import tpu_sc as plsc

B = 16384
F = 128
H1, H2, H3 = 1024, 512, 256

NC, NS = 2, 16
NW = NC * NS
CHUNK = 128
SPLITS = (4096, 12288)
BM = 4096


@functools.cache
def _make_sc_gather(bsub):
    bpw = bsub // NW
    nch = bpw // CHUNK
    mesh = plsc.VectorSubcoreMesh(core_axis_name="c", subcore_axis_name="s")

    @functools.partial(
        pl.kernel,
        mesh=mesh,
        out_type=jax.ShapeDtypeStruct((bsub, 2 * F), jnp.float32),
        scratch_types=[
            pltpu.VMEM((2 * nch, CHUNK), jnp.int32),
            pltpu.VMEM((CHUNK, F), jnp.float32),
            pltpu.VMEM((CHUNK, F), jnp.float32),
            pltpu.SemaphoreType.DMA,
            pltpu.SemaphoreType.DMA,
        ],
    )
    def _sc_gather(uidx, midx, utab, mtab, x, idx_v, buf0, buf1, gsem, wsem):
        wid = lax.axis_index("s") * NC + lax.axis_index("c")
        base = wid * bpw
        pltpu.sync_copy(uidx.at[wid], idx_v.at[pl.ds(0, nch)])
        pltpu.sync_copy(midx.at[wid], idx_v.at[pl.ds(nch, nch)])
        bufs = [buf0, buf1]
        nph = 2 * nch
        tabs = [utab] * nch + [mtab] * nch
        cols = [0] * nch + [F] * nch
        gath = [None, None]
        writes = [None, None]
        gath[0] = pltpu.async_copy(tabs[0].at[idx_v.at[0]], bufs[0], gsem)
        for p in range(nph):
            nxt = p + 1
            if nxt < nph:
                if writes[nxt % 2] is not None:
                    writes[nxt % 2].wait()
                gath[nxt % 2] = pltpu.async_copy(
                    tabs[nxt].at[idx_v.at[nxt]], bufs[nxt % 2], gsem)
            gath[p % 2].wait()
            row = base + (p % nch) * CHUNK
            writes[p % 2] = pltpu.async_copy(
                bufs[p % 2], x.at[pl.ds(row, CHUNK), pl.ds(cols[p], F)], wsem)
        writes[0].wait()
        writes[1].wait()

    return _sc_gather


_NT = (((1,), (1,)), ((), ()))


def _dot_nt(a, w):
    return lax.dot_general(a, w, _NT, preferred_element_type=jnp.float32)


def _mlp_body(x, w1, b1, w2, b2, w3, b3, wf, bf, out):
    bf16 = jnp.bfloat16
    h = jnp.maximum(_dot_nt(x[...].astype(bf16), w1[...]) + b1[...], 0.0).astype(bf16)
    h = jnp.maximum(_dot_nt(h, w2[...]) + b2[...], 0.0).astype(bf16)
    h = jnp.maximum(_dot_nt(h, w3[...]) + b3[...], 0.0).astype(bf16)
    bm = x.shape[0]
    z = _dot_nt(wf[...].astype(bf16), h) + bf[0, 0]
    out[...] = (4.5 * jax.nn.sigmoid(z) + 0.5).reshape(bm // 128, 128)


def _mlp(x, w1, b1, w2, b2, w3, b3, wf, bf, interpret=False):
    bsub = x.shape[0]
    bm = min(BM, bsub)
    const = lambda i: (0, 0)
    return pl.pallas_call(
        _mlp_body,
        grid=(bsub // bm,),
        in_specs=[
            pl.BlockSpec((bm, 2 * F), lambda i: (i, 0)),
            pl.BlockSpec((H1, 2 * F), const),
            pl.BlockSpec((1, H1), const),
            pl.BlockSpec((H2, H1), const),
            pl.BlockSpec((1, H2), const),
            pl.BlockSpec((H3, H2), const),
            pl.BlockSpec((1, H3), const),
            pl.BlockSpec((1, H3), const),
            pl.BlockSpec((1, 1), const),
        ],
        out_specs=pl.BlockSpec((bm // 128, 128), lambda i: (i, 0)),
        out_shape=jax.ShapeDtypeStruct((bsub // 128, 128), jnp.float32),
        interpret=interpret,
    )(x, w1, b1, w2, b2, w3, b3, wf, bf)


def kernel(users, movies, user_table, movie_table, W1, b1, W2, b2, W3, b3, Wf, bf):
    w1 = W1.astype(jnp.bfloat16)
    w2 = W2.astype(jnp.bfloat16)
    w3 = W3.astype(jnp.bfloat16)
    b1r = b1.reshape(1, H1).astype(jnp.bfloat16)
    b2r = b2.reshape(1, H2).astype(jnp.bfloat16)
    b3r = b3.reshape(1, H3).astype(jnp.bfloat16)
    bfr = bf.reshape(1, 1)
    outs = []
    off = 0
    for bsub in SPLITS:
        nch = bsub // NW // CHUNK
        uidx = lax.dynamic_slice_in_dim(users, off, bsub).reshape(NW, nch, CHUNK)
        midx = lax.dynamic_slice_in_dim(movies, off, bsub).reshape(NW, nch, CHUNK)
        x = _make_sc_gather(bsub)(uidx, midx, user_table, movie_table)
        outs.append(_mlp(x, w1, b1r, w2, b2r, w3, b3r, Wf, bfr))
        off += bsub
    return jnp.concatenate(outs, axis=0).reshape(B, 1)


# speedup vs baseline: 1.0276x; 1.0276x over previous
import functools

import jax
import jax.numpy as jnp
from jax import lax
from jax.experimental import pallas as pl
from jax.experimental.pallas import tpu as pltpu
from jax.experimental.pallas import tpu_sc as plsc

B = 16384
F = 128
H1, H2, H3 = 1024, 512, 256

NC, NS = 2, 16
NW = NC * NS
CHUNK = 128
SPLITS = (8192, 8192)
BM = 4096


@functools.cache
def _make_sc_gather(bsub):
    bpw = bsub // NW
    nch = bpw // CHUNK
    mesh = plsc.VectorSubcoreMesh(core_axis_name="c", subcore_axis_name="s")

    @functools.partial(
        pl.kernel,
        mesh=mesh,
        out_type=jax.ShapeDtypeStruct((bsub, 2 * F), jnp.float32),
        scratch_types=[
            pltpu.VMEM((2 * nch, CHUNK), jnp.int32),
            pltpu.VMEM((CHUNK, F), jnp.float32),
            pltpu.VMEM((CHUNK, F), jnp.float32),
            pltpu.SemaphoreType.DMA,
            pltpu.SemaphoreType.DMA,
        ],
    )
    def _sc_gather(uidx, midx, utab, mtab, x, idx_v, buf0, buf1, gsem, wsem):
        wid = lax.axis_index("s") * NC + lax.axis_index("c")
        base = wid * bpw
        pltpu.sync_copy(uidx.at[wid], idx_v.at[pl.ds(0, nch)])
        pltpu.sync_copy(midx.at[wid], idx_v.at[pl.ds(nch, nch)])
        bufs = [buf0, buf1]
        nph = 2 * nch
        tabs = [utab] * nch + [mtab] * nch
        cols = [0] * nch + [F] * nch
        gath = [None, None]
        writes = [None, None]
        gath[0] = pltpu.async_copy(tabs[0].at[idx_v.at[0]], bufs[0], gsem)
        for p in range(nph):
            nxt = p + 1
            if nxt < nph:
                if writes[nxt % 2] is not None:
                    writes[nxt % 2].wait()
                gath[nxt % 2] = pltpu.async_copy(
                    tabs[nxt].at[idx_v.at[nxt]], bufs[nxt % 2], gsem)
            gath[p % 2].wait()
            row = base + (p % nch) * CHUNK
            writes[p % 2] = pltpu.async_copy(
                bufs[p % 2], x.at[pl.ds(row, CHUNK), pl.ds(cols[p], F)], wsem)
        writes[0].wait()
        writes[1].wait()

    return _sc_gather


_NT = (((1,), (1,)), ((), ()))


def _dot_nt(a, w):
    return lax.dot_general(a, w, _NT, preferred_element_type=jnp.float32)


def _mlp_body(x, w1, b1, w2, b2, w3, b3, wf, bf, out):
    bf16 = jnp.bfloat16
    h = jnp.maximum(_dot_nt(x[...].astype(bf16), w1[...]) + b1[...], 0.0).astype(bf16)
    h = jnp.maximum(_dot_nt(h, w2[...]) + b2[...], 0.0).astype(bf16)
    h = jnp.maximum(_dot_nt(h, w3[...]) + b3[...], 0.0).astype(bf16)
    bm = x.shape[0]
    z = _dot_nt(wf[...].astype(bf16), h) + bf[0, 0]
    out[...] = (4.5 * jax.nn.sigmoid(z) + 0.5).reshape(bm // 128, 128)


def _mlp(x, w1, b1, w2, b2, w3, b3, wf, bf, interpret=False):
    bsub = x.shape[0]
    bm = min(BM, bsub)
    const = lambda i: (0, 0)
    return pl.pallas_call(
        _mlp_body,
        grid=(bsub // bm,),
        in_specs=[
            pl.BlockSpec((bm, 2 * F), lambda i: (i, 0)),
            pl.BlockSpec((H1, 2 * F), const),
            pl.BlockSpec((1, H1), const),
            pl.BlockSpec((H2, H1), const),
            pl.BlockSpec((1, H2), const),
            pl.BlockSpec((H3, H2), const),
            pl.BlockSpec((1, H3), const),
            pl.BlockSpec((1, H3), const),
            pl.BlockSpec((1, 1), const),
        ],
        out_specs=pl.BlockSpec((bm // 128, 128), lambda i: (i, 0)),
        out_shape=jax.ShapeDtypeStruct((bsub // 128, 128), jnp.float32),
        interpret=interpret,
    )(x, w1, b1, w2, b2, w3, b3, wf, bf)


def kernel(users, movies, user_table, movie_table, W1, b1, W2, b2, W3, b3, Wf, bf):
    w1 = W1.astype(jnp.bfloat16)
    w2 = W2.astype(jnp.bfloat16)
    w3 = W3.astype(jnp.bfloat16)
    b1r = b1.reshape(1, H1).astype(jnp.bfloat16)
    b2r = b2.reshape(1, H2).astype(jnp.bfloat16)
    b3r = b3.reshape(1, H3).astype(jnp.bfloat16)
    bfr = bf.reshape(1, 1)
    outs = []
    off = 0
    for bsub in SPLITS:
        nch = bsub // NW // CHUNK
        uidx = lax.dynamic_slice_in_dim(users, off, bsub).reshape(NW, nch, CHUNK)
        midx = lax.dynamic_slice_in_dim(movies, off, bsub).reshape(NW, nch, CHUNK)
        x = _make_sc_gather(bsub)(uidx, midx, user_table, movie_table)
        outs.append(_mlp(x, w1, b1r, w2, b2r, w3, b3r, Wf, bfr))
        off += bsub
    return jnp.concatenate(outs, axis=0).reshape(B, 1)
